# Initial kernel scaffold; baseline (speedup 1.0000x reference)
import jax
import jax.numpy as jnp
from jax.experimental import pallas as pl


def kernel(table, indices):
    raise NotImplementedError("write your pallas kernel here")


# speedup vs baseline: 1.2815x; 1.2815x over previous
import functools

import jax
import jax.numpy as jnp
from jax import lax
from jax.experimental import pallas as pl
from jax.experimental.pallas import tpu as pltpu
from jax.experimental.pallas import tpu_sc as plsc

_NC = 2
_NS = 16
_NW = _NC * _NS
_CH = 128


@functools.lru_cache(maxsize=None)
def _make_gather(B, D):
    n_chunks = B // (_NW * _CH)
    mesh = plsc.VectorSubcoreMesh(
        core_axis_name="c", subcore_axis_name="s",
        num_cores=_NC, num_subcores=_NS)

    @functools.partial(
        pl.kernel,
        out_type=jax.ShapeDtypeStruct((B, D), jnp.float32),
        mesh=mesh,
        scratch_types=[
            pltpu.VMEM((n_chunks, _CH), jnp.int32),
            pltpu.VMEM((_CH, D), jnp.float32),
            pltpu.VMEM((_CH, D), jnp.float32),
            pltpu.SemaphoreType.DMA,
            pltpu.SemaphoreType.DMA,
        ],
    )
    def gather_kernel(table_hbm, idx_hbm, out_hbm, idx_v, buf0, buf1,
                      sem0, sem1):
        wid = lax.axis_index("s") * _NC + lax.axis_index("c")
        base = wid * (n_chunks * _CH)
        pltpu.sync_copy(idx_hbm.at[wid], idx_v)
        bufs = (buf0, buf1)
        sems = (sem0, sem1)

        def start(c, b):
            pltpu.async_copy(table_hbm.at[idx_v.at[c]], bufs[b], sems[b])

        def wait(b):
            pltpu.make_async_copy(
                table_hbm.at[idx_v.at[0]], bufs[b], sems[b]).wait()

        start(0, 0)
        start(1, 1)

        def loop_body(g, carry):
            for b in range(2):
                c = g * 2 + b
                wait(b)
                pltpu.sync_copy(bufs[b], out_hbm.at[pl.ds(base + c * _CH, _CH)])
                nxt = c + 2

                @pl.when(nxt < n_chunks)
                def _():
                    start(nxt, b)
            return carry

        lax.fori_loop(0, n_chunks // 2, loop_body, 0)

    return gather_kernel


def kernel(table, indices):
    B = indices.size
    D = table.shape[1]
    idx = indices.reshape(_NW, B // (_NW * _CH), _CH)
    out = _make_gather(B, D)(table, idx)
    return out.reshape(indices.shape + (D,))


# speedup vs baseline: 1.3064x; 1.0195x over previous
import functools

import jax
import jax.numpy as jnp
from jax import lax
from jax.experimental import pallas as pl
from jax.experimental.pallas import tpu as pltpu
from jax.experimental.pallas import tpu_sc as plsc

_NC = 2
_NS = 16
_NW = _NC * _NS
_CH = 104
_NBUF = 4


@functools.lru_cache(maxsize=None)
def _make_gather(B, D):
    n_chunks = B // (_NW * _CH)
    assert n_chunks % _NBUF == 0
    mesh = plsc.VectorSubcoreMesh(
        core_axis_name="c", subcore_axis_name="s",
        num_cores=_NC, num_subcores=_NS)

    @functools.partial(
        pl.kernel,
        out_type=jax.ShapeDtypeStruct((B, D), jnp.float32),
        mesh=mesh,
        scratch_types=[
            pltpu.VMEM((n_chunks, _CH), jnp.int32),
            [pltpu.VMEM((_CH, D), jnp.float32)] * _NBUF,
            [pltpu.SemaphoreType.DMA] * _NBUF,
            [pltpu.SemaphoreType.DMA] * _NBUF,
        ],
    )
    def gather_kernel(table_hbm, idx_hbm, out_hbm, idx_v, bufs,
                      gsems, wsems):
        wid = lax.axis_index("s") * _NC + lax.axis_index("c")
        base = wid * (n_chunks * _CH)
        pltpu.sync_copy(idx_hbm.at[wid], idx_v)

        def start_gather(c, b):
            pltpu.async_copy(table_hbm.at[idx_v.at[c]], bufs[b], gsems[b])

        def wait_gather(b):
            pltpu.make_async_copy(
                table_hbm.at[idx_v.at[0]], bufs[b], gsems[b]).wait()

        def out_slice(c):
            return out_hbm.at[pl.ds(base + c * _CH, _CH)]

        def start_write(c, b):
            pltpu.async_copy(bufs[b], out_slice(c), wsems[b])

        def wait_write(b):
            pltpu.make_async_copy(bufs[b], out_slice(0), wsems[b]).wait()

        for b in range(_NBUF):
            start_gather(b, b)

        def loop_body(p, carry):
            for b in range(_NBUF):
                c = p * _NBUF + b
                wait_gather(b)
                start_write(c, b)
                nxt = c + _NBUF

                @pl.when(nxt < n_chunks)
                def _():
                    wait_write(b)
                    start_gather(nxt, b)
            return carry

        lax.fori_loop(0, n_chunks // _NBUF, loop_body, 0)
        for b in range(_NBUF):
            wait_write(b)

    return gather_kernel


def kernel(table, indices):
    B = indices.size
    D = table.shape[1]
    idx = indices.reshape(_NW, B // (_NW * _CH), _CH)
    out = _make_gather(B, D)(table, idx)
    return out.reshape(indices.shape + (D,))


# speedup vs baseline: 1.9456x; 1.4893x over previous
import functools

import jax
import jax.numpy as jnp
from jax import lax
from jax.experimental import pallas as pl
from jax.experimental.pallas import tpu as pltpu
from jax.experimental.pallas import tpu_sc as plsc

_NC = 2
_NS = 16
_NW = _NC * _NS
_CH = 104
_NBUF = 4


@functools.lru_cache(maxsize=None)
def _make_gather(B, D, out_rows):
    n_chunks = B // (_NW * _CH)
    assert n_chunks % _NBUF == 0
    mesh = plsc.VectorSubcoreMesh(
        core_axis_name="c", subcore_axis_name="s",
        num_cores=_NC, num_subcores=_NS)

    @functools.partial(
        pl.kernel,
        out_type=jax.ShapeDtypeStruct((out_rows, D), jnp.float32),
        mesh=mesh,
        scratch_types=[
            pltpu.VMEM((n_chunks, _CH), jnp.int32),
            pltpu.VMEM((n_chunks, _CH), jnp.int32),
            [pltpu.VMEM((_CH, D), jnp.float32)] * _NBUF,
            [pltpu.SemaphoreType.DMA] * _NBUF,
            [pltpu.SemaphoreType.DMA] * _NBUF,
        ],
    )
    def gather_kernel(table_hbm, idx_hbm, dst_hbm, out_hbm, idx_v, dst_v,
                      bufs, gsems, wsems):
        wid = lax.axis_index("s") * _NC + lax.axis_index("c")
        pltpu.sync_copy(idx_hbm.at[wid], idx_v)
        pltpu.sync_copy(dst_hbm.at[wid], dst_v)

        def start_gather(c, b):
            pltpu.async_copy(table_hbm.at[idx_v.at[c]], bufs[b], gsems[b])

        def wait_gather(b):
            pltpu.make_async_copy(
                table_hbm.at[idx_v.at[0]], bufs[b], gsems[b]).wait()

        def start_write(c, b):
            pltpu.async_copy(bufs[b], out_hbm.at[dst_v.at[c]], wsems[b])

        def wait_write(b):
            pltpu.make_async_copy(
                bufs[b], out_hbm.at[dst_v.at[0]], wsems[b]).wait()

        for b in range(_NBUF):
            start_gather(b, b)

        def loop_body(p, carry):
            for b in range(_NBUF):
                c = p * _NBUF + b
                wait_gather(b)
                start_write(c, b)
                nxt = c + _NBUF

                @pl.when(nxt < n_chunks)
                def _():
                    wait_write(b)
                    start_gather(nxt, b)
            return carry

        lax.fori_loop(0, n_chunks // _NBUF, loop_body, 0)
        for b in range(_NBUF):
            wait_write(b)

    return gather_kernel


def kernel(table, indices):
    N, F = indices.shape
    D = table.shape[1]
    B = N * F
    P = (F + 7) // 8 * 8
    idx = indices.reshape(_NW, B // (_NW * _CH), _CH)
    r = jnp.arange(B, dtype=jnp.int32)
    dst = ((r // F) * P + r % F).reshape(idx.shape)
    out = _make_gather(B, D, N * P)(table, idx, dst)
    return out.reshape(N, P, D)[:, :F, :]


# speedup vs baseline: 2.0548x; 1.0562x over previous
import functools

import jax
import jax.numpy as jnp
from jax import lax
from jax.experimental import pallas as pl
from jax.experimental.pallas import tpu as pltpu
from jax.experimental.pallas import tpu_sc as plsc

_NC = 2
_NS = 16
_NW = _NC * _NS
_CB = 4
_NBUF = 4


@functools.lru_cache(maxsize=None)
def _make_gather(N, F, D):
    bpw = N // _NW
    n_chunks = bpw // _CB
    ch = _CB * F
    assert n_chunks % _NBUF == 0 and ch <= 128
    mesh = plsc.VectorSubcoreMesh(
        core_axis_name="c", subcore_axis_name="s",
        num_cores=_NC, num_subcores=_NS)

    @functools.partial(
        pl.kernel,
        out_type=jax.ShapeDtypeStruct((N, F, D), jnp.float32),
        mesh=mesh,
        scratch_types=[
            pltpu.VMEM((n_chunks, ch), jnp.int32),
            [pltpu.VMEM((ch, D), jnp.float32)] * _NBUF,
            [pltpu.SemaphoreType.DMA] * _NBUF,
            [pltpu.SemaphoreType.DMA] * _NBUF,
        ],
    )
    def gather_kernel(table_hbm, idx_hbm, out_hbm, idx_v, bufs,
                      gsems, wsems):
        wid = lax.axis_index("s") * _NC + lax.axis_index("c")
        b_base = wid * bpw
        pltpu.sync_copy(idx_hbm.at[wid], idx_v)

        def start_gather(c, b):
            pltpu.async_copy(table_hbm.at[idx_v.at[c]], bufs[b], gsems[b])

        def wait_gather(b):
            pltpu.make_async_copy(
                table_hbm.at[idx_v.at[0]], bufs[b], gsems[b]).wait()

        def start_write(c, b):
            b0 = b_base + c * _CB
            for j in range(_CB):
                pltpu.async_copy(
                    bufs[b].at[pl.ds(j * F, F)], out_hbm.at[b0 + j],
                    wsems[b])

        def wait_write(b):
            for j in range(_CB):
                pltpu.make_async_copy(
                    bufs[b].at[pl.ds(0, F)], out_hbm.at[0], wsems[b]).wait()

        for b in range(_NBUF):
            start_gather(b, b)

        def loop_body(p, carry):
            for b in range(_NBUF):
                c = p * _NBUF + b
                wait_gather(b)
                start_write(c, b)
                nxt = c + _NBUF

                @pl.when(nxt < n_chunks)
                def _():
                    wait_write(b)
                    start_gather(nxt, b)
            return carry

        lax.fori_loop(0, n_chunks // _NBUF, loop_body, 0)
        for b in range(_NBUF):
            wait_write(b)

    return gather_kernel


def kernel(table, indices):
    N, F = indices.shape
    D = table.shape[1]
    ch = _CB * F
    idx = indices.reshape(_NW, N * F // (_NW * ch), ch)
    return _make_gather(N, F, D)(table, idx)
